# baseline (speedup 1.0000x reference)
import functools

import jax
import jax.numpy as jnp
from jax import lax
from jax.experimental import pallas as pl
from jax.experimental.pallas import tpu as pltpu
from jax.experimental.pallas import tpu_sc as plsc

N = 10000
E = 160000
D = 256
NPAD = 10240
NTILES = 32
PT = NPAD // NTILES
GARBAGE = PT

SCHUNK = 800
NSCHUNK = E // SCHUNK
FLUSH = 1024
TAILFLUSH = 1104
PENDC = 1936
TRASH = PENDC
EC = E + FLUSH + TAILFLUSH

G = 48

_PARAMS = pltpu.CompilerParams(needs_layout_passes=False)


def _wid():
    return lax.axis_index("s") * 2 + lax.axis_index("c")


def _sc_bucket_body(src, dst, bsrc, bdl, counts,
                    srcA_v, dstA_v, srcB_v, dstB_v, psrc_v, pdl_v, cnt_v,
                    semA, semB):
    w = _wid()
    base = w * PT
    rowoff = w * EC

    def start(j, sbuf, dbuf, sem):
        off = pl.multiple_of(j * SCHUNK, 8)
        pltpu.make_async_copy(src.at[pl.ds(off, SCHUNK)], sbuf, sem).start()
        pltpu.make_async_copy(dst.at[pl.ds(off, SCHUNK)], dbuf, sem).start()

    def wait(sbuf, dbuf, sem):
        pltpu.make_async_copy(src.at[pl.ds(0, SCHUNK)], sbuf, sem).wait()
        pltpu.make_async_copy(dst.at[pl.ds(0, SCHUNK)], dbuf, sem).wait()

    def process(sbuf, dbuf, carry):
        f, gcnt = carry
        fvec0 = jnp.full((16,), 0, jnp.int32) + f

        def group(g, fvec):
            d = dbuf[pl.ds(g * 16, 16)]
            sv = sbuf[pl.ds(g * 16, 16)]
            dl = d - base
            ok = (dl >= 0) & (dl < PT)
            oki = ok.astype(jnp.int32)
            incl = plsc.cumsum(oki)
            popc = plsc.all_reduce_population_count(ok)
            pos = jnp.where(ok, fvec + (incl - oki), TRASH)
            plsc.store_scatter(psrc_v, [pos], sv)
            plsc.store_scatter(pdl_v, [pos], dl)
            return fvec + popc

        fvec = lax.fori_loop(0, SCHUNK // 16, group, fvec0)
        f = fvec[0]

        def do_flush(carry):
            f, gcnt = carry
            foff = pl.multiple_of(rowoff + gcnt, 8)
            pltpu.sync_copy(psrc_v.at[pl.ds(0, FLUSH)],
                            bsrc.at[pl.ds(foff, FLUSH)])
            pltpu.sync_copy(pdl_v.at[pl.ds(0, FLUSH)],
                            bdl.at[pl.ds(foff, FLUSH)])
            nrem = (f - FLUSH + 15) // 16

            def shift(g, c):
                psrc_v[pl.ds(g * 16, 16)] = psrc_v[pl.ds(FLUSH + g * 16, 16)]
                pdl_v[pl.ds(g * 16, 16)] = pdl_v[pl.ds(FLUSH + g * 16, 16)]
                return c

            lax.fori_loop(0, nrem, shift, 0)
            return f - FLUSH, gcnt + FLUSH

        return lax.cond(f >= FLUSH, do_flush, lambda c: c, (f, gcnt))

    start(0, srcA_v, dstA_v, semA)

    def scan_pair(j2, carry):
        start(2 * j2 + 1, srcB_v, dstB_v, semB)
        wait(srcA_v, dstA_v, semA)
        carry = process(srcA_v, dstA_v, carry)

        @pl.when(2 * j2 + 2 < NSCHUNK)
        def _():
            start(2 * j2 + 2, srcA_v, dstA_v, semA)

        wait(srcB_v, dstB_v, semB)
        return process(srcB_v, dstB_v, carry)

    f, gcnt = lax.fori_loop(0, NSCHUNK // 2, scan_pair, (0, 0))

    zeros16 = jnp.full((16,), 0, jnp.int32)
    garb16 = jnp.full((16,), GARBAGE, jnp.int32)
    for g in range(5):
        psrc_v[pl.ds(f + g * 16, 16)] = zeros16
        pdl_v[pl.ds(f + g * 16, 16)] = garb16
    toff = pl.multiple_of(rowoff + gcnt, 8)
    pltpu.sync_copy(psrc_v.at[pl.ds(0, TAILFLUSH)],
                    bsrc.at[pl.ds(toff, TAILFLUSH)])
    pltpu.sync_copy(pdl_v.at[pl.ds(0, TAILFLUSH)],
                    bdl.at[pl.ds(toff, TAILFLUSH)])
    cnt_v[pl.ds(0, 16)] = zeros16 + (gcnt + f)
    pltpu.sync_copy(cnt_v, counts.at[pl.ds(w * 16, 16)])


_sc_bucket = pl.kernel(
    _sc_bucket_body,
    out_type=(
        jax.ShapeDtypeStruct((NTILES * EC,), jnp.int32),
        jax.ShapeDtypeStruct((NTILES * EC,), jnp.int32),
        jax.ShapeDtypeStruct((NTILES * 16,), jnp.int32),
    ),
    mesh=plsc.VectorSubcoreMesh(core_axis_name="c", subcore_axis_name="s"),
    compiler_params=_PARAMS,
    scratch_types=[
        pltpu.VMEM((SCHUNK,), jnp.int32),
        pltpu.VMEM((SCHUNK,), jnp.int32),
        pltpu.VMEM((SCHUNK,), jnp.int32),
        pltpu.VMEM((SCHUNK,), jnp.int32),
        pltpu.VMEM((PENDC + 16,), jnp.int32),
        pltpu.VMEM((PENDC + 16,), jnp.int32),
        pltpu.VMEM((16,), jnp.int32),
        pltpu.SemaphoreType.DMA,
        pltpu.SemaphoreType.DMA,
    ],
)


WB = 21
WIN = WB * G


def _sc_agg_body(feat, bsrc, bdl, counts, agg_out,
                 srcw_v, dlw_v, cnt_v, rows0_v, rows1_v, rows2_v, acc_v,
                 deg_v, sem0, sem1, sem2):
    w = _wid()
    rowoff = w * EC

    zrow = jnp.full((16,), 0.0, jnp.float32)

    def zacc(r, c):
        for k in range(D // 16):
            acc_v[r, pl.ds(k * 16, 16)] = zrow
        return c

    lax.fori_loop(0, PT + 8, zacc, 0)
    for g in range((PT + 32) // 16):
        deg_v[pl.ds(g * 16, 16)] = zrow

    pltpu.sync_copy(counts.at[pl.ds(w * 16, 16)], cnt_v)
    cnt = cnt_v[pl.ds(0, 16)][0]
    nbatch = (cnt + G - 1) // G
    nwin = (nbatch + WB - 1) // WB
    onehot = jnp.where(lax.iota(jnp.int32, 16) == 0, 1.0, 0.0)
    iota16 = lax.iota(jnp.int32, 16)
    col_idx = [iota16 + 16 * k for k in range(D // 16)]

    def gdesc(ib, buf, sem):
        off = pl.multiple_of(ib * G, 8)
        return pltpu.make_async_copy(feat.at[srcw_v.at[pl.ds(off, G)]],
                                     buf, sem)

    def process(ib, buf):
        def edge2(e2, c):
            e = 2 * e2
            ebase = ib * G + e
            eidxa = jnp.full((16,), 0, jnp.int32) + ebase
            dlba = plsc.load_gather(dlw_v, [eidxa])
            dlbb = plsc.load_gather(dlw_v, [eidxa + 1])
            plsc.addupdate_scatter(deg_v, [dlba + iota16], onehot)
            plsc.addupdate_scatter(deg_v, [dlbb + iota16], onehot)
            for k in range(D // 16):
                va = buf[e, pl.ds(k * 16, 16)]
                vb = buf[e + 1, pl.ds(k * 16, 16)]
                plsc.addupdate_scatter(acc_v, [dlba, col_idx[k]], va)
                plsc.addupdate_scatter(acc_v, [dlbb, col_idx[k]], vb)
            return c

        lax.fori_loop(0, G // 2, edge2, 0)

    def window(wi, carry):
        woff = pl.multiple_of(rowoff + wi * WIN, 8)
        pltpu.sync_copy(bsrc.at[pl.ds(woff, WIN)], srcw_v)
        pltpu.sync_copy(bdl.at[pl.ds(woff, WIN)], dlw_v.at[pl.ds(0, WIN)])
        nb = jnp.minimum(nbatch - wi * WB, WB)

        gdesc(0, rows0_v, sem0).start()

        @pl.when(nb >= 2)
        def _():
            gdesc(1, rows1_v, sem1).start()

        def triple(t, c):
            i = 3 * t
            gdesc(i, rows0_v, sem0).wait()

            @pl.when(i + 2 < nb)
            def _():
                gdesc(i + 2, rows2_v, sem2).start()

            process(i, rows0_v)
            gdesc(i + 1, rows1_v, sem1).wait()

            @pl.when(i + 3 < nb)
            def _():
                gdesc(i + 3, rows0_v, sem0).start()

            process(i + 1, rows1_v)
            gdesc(i + 2, rows2_v, sem2).wait()

            @pl.when(i + 4 < nb)
            def _():
                gdesc(i + 4, rows1_v, sem1).start()

            process(i + 2, rows2_v)
            return c

        ntrip = nb // 3
        lax.fori_loop(0, ntrip, triple, 0)
        rem = nb - 3 * ntrip

        @pl.when(rem >= 1)
        def _():
            gdesc(3 * ntrip, rows0_v, sem0).wait()
            process(3 * ntrip, rows0_v)

        @pl.when(rem == 2)
        def _():
            gdesc(3 * ntrip + 1, rows1_v, sem1).wait()
            process(3 * ntrip + 1, rows1_v)

        return carry

    lax.fori_loop(0, nwin, window, 0)

    def norm(r, c):
        recv = 1.0 / jnp.maximum(deg_v[pl.ds(r, 16)], 1.0)
        rec = recv[0]
        for k in range(D // 16):
            acc_v[r, pl.ds(k * 16, 16)] = acc_v[r, pl.ds(k * 16, 16)] * rec
        return c

    lax.fori_loop(0, PT, norm, 0)
    pltpu.sync_copy(acc_v.at[pl.ds(0, PT)], agg_out.at[pl.ds(w * PT, PT)])


_sc_agg = pl.kernel(
    _sc_agg_body,
    out_type=jax.ShapeDtypeStruct((NPAD, D), jnp.float32),
    mesh=plsc.VectorSubcoreMesh(core_axis_name="c", subcore_axis_name="s"),
    compiler_params=_PARAMS,
    scratch_types=[
        pltpu.VMEM((WIN,), jnp.int32),
        pltpu.VMEM((WIN + 16,), jnp.int32),
        pltpu.VMEM((16,), jnp.int32),
        pltpu.VMEM((G, D), jnp.float32),
        pltpu.VMEM((G, D), jnp.float32),
        pltpu.VMEM((G, D), jnp.float32),
        pltpu.VMEM((PT + 8, D), jnp.float32),
        pltpu.VMEM((PT + 32,), jnp.float32),
        pltpu.SemaphoreType.DMA,
        pltpu.SemaphoreType.DMA,
        pltpu.SemaphoreType.DMA,
    ],
)


def _dense_body(agg_ref, x_ref, wl_ref, wr_ref, b_ref, o_ref, *, relu):
    acc = lax.dot_general(agg_ref[...], wl_ref[...], (((1,), (1,)), ((), ())),
                          preferred_element_type=jnp.float32)
    acc = acc + lax.dot_general(x_ref[...], wr_ref[...], (((1,), (1,)), ((), ())),
                                preferred_element_type=jnp.float32)
    acc = acc + b_ref[...]
    if relu:
        acc = jnp.maximum(acc, 0.0)
    o_ref[...] = acc


def _dense(aggn, x, W_l, W_r, b, relu):
    BR = 1024
    grid = NPAD // BR
    return pl.pallas_call(
        functools.partial(_dense_body, relu=relu),
        grid=(grid,),
        in_specs=[
            pl.BlockSpec((BR, D), lambda i: (i, 0)),
            pl.BlockSpec((BR, D), lambda i: (i, 0)),
            pl.BlockSpec((D, D), lambda i: (0, 0)),
            pl.BlockSpec((D, D), lambda i: (0, 0)),
            pl.BlockSpec((1, D), lambda i: (0, 0)),
        ],
        out_specs=pl.BlockSpec((BR, D), lambda i: (i, 0)),
        out_shape=jax.ShapeDtypeStruct((NPAD, D), jnp.float32),
    )(aggn, x, W_l, W_r, b)


def kernel(x, edge_index, W1_l, b1_l, W1_r, W2_l, b2_l, W2_r):
    x = x.astype(jnp.float32)
    ei = edge_index.astype(jnp.int32)
    src = ei[0]
    dst = ei[1]
    xpad = jnp.zeros((NPAD, D), jnp.float32).at[:N].set(x)
    b1 = b1_l.reshape(1, D)
    b2 = b2_l.reshape(1, D)

    bsrc, bdl, counts = _sc_bucket(src, dst)
    agg1 = _sc_agg(xpad, bsrc, bdl, counts)
    h = _dense(agg1, xpad, W1_l, W1_r, b1, relu=True)
    agg2 = _sc_agg(h, bsrc, bdl, counts)
    out = _dense(agg2, h, W2_l, W2_r, b2, relu=False)
    return out[:N]


# speedup vs baseline: 1.1828x; 1.1828x over previous
import functools

import jax
import jax.numpy as jnp
from jax import lax
from jax.experimental import pallas as pl
from jax.experimental.pallas import tpu as pltpu
from jax.experimental.pallas import tpu_sc as plsc

N = 10000
E = 160000
D = 256
NPAD = 10240
NTILES = 32
PT = NPAD // NTILES
GARBAGE = PT

SCHUNK = 800
NSCHUNK = E // SCHUNK
FLUSH = 1024
TAILFLUSH = 1104
PENDC = 1936
TRASH = PENDC
EC = E + FLUSH + TAILFLUSH

G = 48

_PARAMS = pltpu.CompilerParams(needs_layout_passes=False)


def _wid():
    return lax.axis_index("s") * 2 + lax.axis_index("c")


def _sc_bucket_body(src, dst, bsrc, bdl, counts,
                    srcA_v, dstA_v, srcB_v, dstB_v, psrc_v, pdl_v, cnt_v,
                    semA, semB):
    w = _wid()
    base = w * PT
    rowoff = w * EC

    def start(j, sbuf, dbuf, sem):
        off = pl.multiple_of(j * SCHUNK, 8)
        pltpu.make_async_copy(src.at[pl.ds(off, SCHUNK)], sbuf, sem).start()
        pltpu.make_async_copy(dst.at[pl.ds(off, SCHUNK)], dbuf, sem).start()

    def wait(sbuf, dbuf, sem):
        pltpu.make_async_copy(src.at[pl.ds(0, SCHUNK)], sbuf, sem).wait()
        pltpu.make_async_copy(dst.at[pl.ds(0, SCHUNK)], dbuf, sem).wait()

    def process(sbuf, dbuf, carry):
        f, gcnt = carry
        fvec0 = jnp.full((16,), 0, jnp.int32) + f

        def group(g, fvec):
            d = dbuf[pl.ds(g * 16, 16)]
            sv = sbuf[pl.ds(g * 16, 16)]
            dl = d - base
            ok = (dl >= 0) & (dl < PT)
            oki = ok.astype(jnp.int32)
            incl = plsc.cumsum(oki)
            popc = plsc.all_reduce_population_count(ok)
            pos = jnp.where(ok, fvec + (incl - oki), TRASH)
            plsc.store_scatter(psrc_v, [pos], sv)
            plsc.store_scatter(pdl_v, [pos], dl)
            return fvec + popc

        fvec = lax.fori_loop(0, SCHUNK // 16, group, fvec0)
        f = fvec[0]

        def do_flush(carry):
            f, gcnt = carry
            foff = pl.multiple_of(rowoff + gcnt, 8)
            pltpu.sync_copy(psrc_v.at[pl.ds(0, FLUSH)],
                            bsrc.at[pl.ds(foff, FLUSH)])
            pltpu.sync_copy(pdl_v.at[pl.ds(0, FLUSH)],
                            bdl.at[pl.ds(foff, FLUSH)])
            nrem = (f - FLUSH + 15) // 16

            def shift(g, c):
                psrc_v[pl.ds(g * 16, 16)] = psrc_v[pl.ds(FLUSH + g * 16, 16)]
                pdl_v[pl.ds(g * 16, 16)] = pdl_v[pl.ds(FLUSH + g * 16, 16)]
                return c

            lax.fori_loop(0, nrem, shift, 0)
            return f - FLUSH, gcnt + FLUSH

        return lax.cond(f >= FLUSH, do_flush, lambda c: c, (f, gcnt))

    start(0, srcA_v, dstA_v, semA)

    def scan_pair(j2, carry):
        start(2 * j2 + 1, srcB_v, dstB_v, semB)
        wait(srcA_v, dstA_v, semA)
        carry = process(srcA_v, dstA_v, carry)

        @pl.when(2 * j2 + 2 < NSCHUNK)
        def _():
            start(2 * j2 + 2, srcA_v, dstA_v, semA)

        wait(srcB_v, dstB_v, semB)
        return process(srcB_v, dstB_v, carry)

    f, gcnt = lax.fori_loop(0, NSCHUNK // 2, scan_pair, (0, 0))

    zeros16 = jnp.full((16,), 0, jnp.int32)
    garb16 = jnp.full((16,), GARBAGE, jnp.int32)
    for g in range(5):
        psrc_v[pl.ds(f + g * 16, 16)] = zeros16
        pdl_v[pl.ds(f + g * 16, 16)] = garb16
    toff = pl.multiple_of(rowoff + gcnt, 8)
    pltpu.sync_copy(psrc_v.at[pl.ds(0, TAILFLUSH)],
                    bsrc.at[pl.ds(toff, TAILFLUSH)])
    pltpu.sync_copy(pdl_v.at[pl.ds(0, TAILFLUSH)],
                    bdl.at[pl.ds(toff, TAILFLUSH)])
    cnt_v[pl.ds(0, 16)] = zeros16 + (gcnt + f)
    pltpu.sync_copy(cnt_v, counts.at[pl.ds(w * 16, 16)])


_sc_bucket = pl.kernel(
    _sc_bucket_body,
    out_type=(
        jax.ShapeDtypeStruct((NTILES * EC,), jnp.int32),
        jax.ShapeDtypeStruct((NTILES * EC,), jnp.int32),
        jax.ShapeDtypeStruct((NTILES * 16,), jnp.int32),
    ),
    mesh=plsc.VectorSubcoreMesh(core_axis_name="c", subcore_axis_name="s"),
    compiler_params=_PARAMS,
    scratch_types=[
        pltpu.VMEM((SCHUNK,), jnp.int32),
        pltpu.VMEM((SCHUNK,), jnp.int32),
        pltpu.VMEM((SCHUNK,), jnp.int32),
        pltpu.VMEM((SCHUNK,), jnp.int32),
        pltpu.VMEM((PENDC + 16,), jnp.int32),
        pltpu.VMEM((PENDC + 16,), jnp.int32),
        pltpu.VMEM((16,), jnp.int32),
        pltpu.SemaphoreType.DMA,
        pltpu.SemaphoreType.DMA,
    ],
)


WB = 21
WIN = WB * G


def _sc_agg_body(feat, bsrc, bdl, counts, agg_out,
                 srcw_v, dlw_v, cnt_v, rows0_v, rows1_v, rows2_v, acc_v,
                 deg_v, sem0, sem1, sem2):
    w = _wid()
    rowoff = w * EC

    zrow = jnp.full((16,), 0.0, jnp.float32)

    def zacc(r, c):
        for k in range(D // 16):
            acc_v[r, pl.ds(k * 16, 16)] = zrow
        return c

    lax.fori_loop(0, PT + 8, zacc, 0)
    for g in range((PT + 32) // 16):
        deg_v[pl.ds(g * 16, 16)] = zrow

    pltpu.sync_copy(counts.at[pl.ds(w * 16, 16)], cnt_v)
    cnt = cnt_v[pl.ds(0, 16)][0]
    nbatch = (cnt + G - 1) // G
    nwin = (nbatch + WB - 1) // WB
    onehot = jnp.where(lax.iota(jnp.int32, 16) == 0, 1.0, 0.0)
    iota16 = lax.iota(jnp.int32, 16)
    col_idx = [iota16 + 16 * k for k in range(D // 16)]

    def gdesc(ib, buf, sem):
        off = pl.multiple_of(ib * G, 8)
        return pltpu.make_async_copy(feat.at[srcw_v.at[pl.ds(off, G)]],
                                     buf, sem)

    def process(ib, buf):
        def edge(e, c):
            eidx = jnp.full((16,), 0, jnp.int32) + (ib * G + e)
            dlb = plsc.load_gather(dlw_v, [eidx])
            plsc.addupdate_scatter(deg_v, [dlb + iota16], onehot)
            vals = [buf[e, pl.ds(k * 16, 16)] for k in range(D // 16)]
            for k in range(D // 16):
                plsc.addupdate_scatter(acc_v, [dlb, col_idx[k]], vals[k])
            return c

        lax.fori_loop(0, G, edge, 0)

    def window(wi, carry):
        woff = pl.multiple_of(rowoff + wi * WIN, 8)
        pltpu.sync_copy(bsrc.at[pl.ds(woff, WIN)], srcw_v)
        pltpu.sync_copy(bdl.at[pl.ds(woff, WIN)], dlw_v.at[pl.ds(0, WIN)])
        nb = jnp.minimum(nbatch - wi * WB, WB)

        gdesc(0, rows0_v, sem0).start()

        @pl.when(nb >= 2)
        def _():
            gdesc(1, rows1_v, sem1).start()

        def triple(t, c):
            i = 3 * t
            gdesc(i, rows0_v, sem0).wait()

            @pl.when(i + 2 < nb)
            def _():
                gdesc(i + 2, rows2_v, sem2).start()

            process(i, rows0_v)
            gdesc(i + 1, rows1_v, sem1).wait()

            @pl.when(i + 3 < nb)
            def _():
                gdesc(i + 3, rows0_v, sem0).start()

            process(i + 1, rows1_v)
            gdesc(i + 2, rows2_v, sem2).wait()

            @pl.when(i + 4 < nb)
            def _():
                gdesc(i + 4, rows1_v, sem1).start()

            process(i + 2, rows2_v)
            return c

        ntrip = nb // 3
        lax.fori_loop(0, ntrip, triple, 0)
        rem = nb - 3 * ntrip

        @pl.when(rem >= 1)
        def _():
            gdesc(3 * ntrip, rows0_v, sem0).wait()
            process(3 * ntrip, rows0_v)

        @pl.when(rem == 2)
        def _():
            gdesc(3 * ntrip + 1, rows1_v, sem1).wait()
            process(3 * ntrip + 1, rows1_v)

        return carry

    lax.fori_loop(0, nwin, window, 0)

    def norm(r, c):
        recv = 1.0 / jnp.maximum(deg_v[pl.ds(r, 16)], 1.0)
        rec = recv[0]
        for k in range(D // 16):
            acc_v[r, pl.ds(k * 16, 16)] = acc_v[r, pl.ds(k * 16, 16)] * rec
        return c

    lax.fori_loop(0, PT, norm, 0)
    pltpu.sync_copy(acc_v.at[pl.ds(0, PT)], agg_out.at[pl.ds(w * PT, PT)])


_sc_agg = pl.kernel(
    _sc_agg_body,
    out_type=jax.ShapeDtypeStruct((NPAD, D), jnp.float32),
    mesh=plsc.VectorSubcoreMesh(core_axis_name="c", subcore_axis_name="s"),
    compiler_params=_PARAMS,
    scratch_types=[
        pltpu.VMEM((WIN,), jnp.int32),
        pltpu.VMEM((WIN + 16,), jnp.int32),
        pltpu.VMEM((16,), jnp.int32),
        pltpu.VMEM((G, D), jnp.float32),
        pltpu.VMEM((G, D), jnp.float32),
        pltpu.VMEM((G, D), jnp.float32),
        pltpu.VMEM((PT + 8, D), jnp.float32),
        pltpu.VMEM((PT + 32,), jnp.float32),
        pltpu.SemaphoreType.DMA,
        pltpu.SemaphoreType.DMA,
        pltpu.SemaphoreType.DMA,
    ],
)


def _dense_r_body(x_ref, wr_ref, b_ref, o_ref):
    acc = lax.dot_general(x_ref[...], wr_ref[...], (((1,), (1,)), ((), ())),
                          preferred_element_type=jnp.float32)
    o_ref[...] = acc + b_ref[...]


def _dense_r(x, W_r, b):
    BR = 1024
    return pl.pallas_call(
        _dense_r_body,
        grid=(NPAD // BR,),
        in_specs=[
            pl.BlockSpec((BR, D), lambda i: (i, 0)),
            pl.BlockSpec((D, D), lambda i: (0, 0)),
            pl.BlockSpec((1, D), lambda i: (0, 0)),
        ],
        out_specs=pl.BlockSpec((BR, D), lambda i: (i, 0)),
        out_shape=jax.ShapeDtypeStruct((NPAD, D), jnp.float32),
    )(x, W_r, b)


def _dense_l_body(agg_ref, wl_ref, xr_ref, o_ref, *, relu):
    acc = lax.dot_general(agg_ref[...], wl_ref[...], (((1,), (1,)), ((), ())),
                          preferred_element_type=jnp.float32)
    acc = acc + xr_ref[...]
    if relu:
        acc = jnp.maximum(acc, 0.0)
    o_ref[...] = acc


def _dense_l(aggn, W_l, xr, relu):
    BR = 1024
    return pl.pallas_call(
        functools.partial(_dense_l_body, relu=relu),
        grid=(NPAD // BR,),
        in_specs=[
            pl.BlockSpec((BR, D), lambda i: (i, 0)),
            pl.BlockSpec((D, D), lambda i: (0, 0)),
            pl.BlockSpec((BR, D), lambda i: (i, 0)),
        ],
        out_specs=pl.BlockSpec((BR, D), lambda i: (i, 0)),
        out_shape=jax.ShapeDtypeStruct((NPAD, D), jnp.float32),
    )(aggn, W_l, xr)


def kernel(x, edge_index, W1_l, b1_l, W1_r, W2_l, b2_l, W2_r):
    x = x.astype(jnp.float32)
    ei = edge_index.astype(jnp.int32)
    src = ei[0]
    dst = ei[1]
    xpad = jnp.zeros((NPAD, D), jnp.float32).at[:N].set(x)
    b1 = b1_l.reshape(1, D)
    b2 = b2_l.reshape(1, D)

    bsrc, bdl, counts = _sc_bucket(src, dst)
    xr1 = _dense_r(xpad, W1_r, b1)
    agg1 = _sc_agg(xpad, bsrc, bdl, counts)
    h = _dense_l(agg1, W1_l, xr1, relu=True)
    xr2 = _dense_r(h, W2_r, b2)
    agg2 = _sc_agg(h, bsrc, bdl, counts)
    out = _dense_l(agg2, W2_l, xr2, relu=False)
    return out[:N]


# speedup vs baseline: 1.1872x; 1.0037x over previous
import functools

import jax
import jax.numpy as jnp
from jax import lax
from jax.experimental import pallas as pl
from jax.experimental.pallas import tpu as pltpu
from jax.experimental.pallas import tpu_sc as plsc

N = 10000
E = 160000
D = 256
NPAD = 10240
NTILES = 32
PT = NPAD // NTILES
GARBAGE = PT

SCHUNK = 800
NSCHUNK = E // SCHUNK
FLUSH = 1024
TAILFLUSH = 1104
PENDC = 1936
TRASH = PENDC
EC = E + FLUSH + TAILFLUSH

G = 48

_PARAMS = pltpu.CompilerParams(needs_layout_passes=False)


def _wid():
    return lax.axis_index("s") * 2 + lax.axis_index("c")


def _sc_bucket_body(src, dst, bsrc, bdl, counts,
                    srcA_v, dstA_v, srcB_v, dstB_v, psrc_v, pdl_v, cnt_v,
                    semA, semB):
    w = _wid()
    base = w * PT
    rowoff = w * EC

    def start(j, sbuf, dbuf, sem):
        off = pl.multiple_of(j * SCHUNK, 8)
        pltpu.make_async_copy(src.at[pl.ds(off, SCHUNK)], sbuf, sem).start()
        pltpu.make_async_copy(dst.at[pl.ds(off, SCHUNK)], dbuf, sem).start()

    def wait(sbuf, dbuf, sem):
        pltpu.make_async_copy(src.at[pl.ds(0, SCHUNK)], sbuf, sem).wait()
        pltpu.make_async_copy(dst.at[pl.ds(0, SCHUNK)], dbuf, sem).wait()

    def process(sbuf, dbuf, carry):
        f, gcnt = carry
        fvec0 = jnp.full((16,), 0, jnp.int32) + f

        def group(g, fvec):
            d = dbuf[pl.ds(g * 16, 16)]
            sv = sbuf[pl.ds(g * 16, 16)]
            dl = d - base
            ok = (dl >= 0) & (dl < PT)
            oki = ok.astype(jnp.int32)
            incl = plsc.cumsum(oki)
            popc = plsc.all_reduce_population_count(ok)
            pos = jnp.where(ok, fvec + (incl - oki), TRASH)
            plsc.store_scatter(psrc_v, [pos], sv)
            plsc.store_scatter(pdl_v, [pos], dl)
            return fvec + popc

        fvec = lax.fori_loop(0, SCHUNK // 16, group, fvec0)
        f = fvec[0]

        def do_flush(carry):
            f, gcnt = carry
            foff = pl.multiple_of(rowoff + gcnt, 8)
            pltpu.sync_copy(psrc_v.at[pl.ds(0, FLUSH)],
                            bsrc.at[pl.ds(foff, FLUSH)])
            pltpu.sync_copy(pdl_v.at[pl.ds(0, FLUSH)],
                            bdl.at[pl.ds(foff, FLUSH)])
            nrem = (f - FLUSH + 15) // 16

            def shift(g, c):
                psrc_v[pl.ds(g * 16, 16)] = psrc_v[pl.ds(FLUSH + g * 16, 16)]
                pdl_v[pl.ds(g * 16, 16)] = pdl_v[pl.ds(FLUSH + g * 16, 16)]
                return c

            lax.fori_loop(0, nrem, shift, 0)
            return f - FLUSH, gcnt + FLUSH

        return lax.cond(f >= FLUSH, do_flush, lambda c: c, (f, gcnt))

    start(0, srcA_v, dstA_v, semA)

    def scan_pair(j2, carry):
        start(2 * j2 + 1, srcB_v, dstB_v, semB)
        wait(srcA_v, dstA_v, semA)
        carry = process(srcA_v, dstA_v, carry)

        @pl.when(2 * j2 + 2 < NSCHUNK)
        def _():
            start(2 * j2 + 2, srcA_v, dstA_v, semA)

        wait(srcB_v, dstB_v, semB)
        return process(srcB_v, dstB_v, carry)

    f, gcnt = lax.fori_loop(0, NSCHUNK // 2, scan_pair, (0, 0))

    zeros16 = jnp.full((16,), 0, jnp.int32)
    garb16 = jnp.full((16,), GARBAGE, jnp.int32)
    for g in range(5):
        psrc_v[pl.ds(f + g * 16, 16)] = zeros16
        pdl_v[pl.ds(f + g * 16, 16)] = garb16
    toff = pl.multiple_of(rowoff + gcnt, 8)
    pltpu.sync_copy(psrc_v.at[pl.ds(0, TAILFLUSH)],
                    bsrc.at[pl.ds(toff, TAILFLUSH)])
    pltpu.sync_copy(pdl_v.at[pl.ds(0, TAILFLUSH)],
                    bdl.at[pl.ds(toff, TAILFLUSH)])
    cnt_v[pl.ds(0, 16)] = zeros16 + (gcnt + f)
    pltpu.sync_copy(cnt_v, counts.at[pl.ds(w * 16, 16)])


_sc_bucket = pl.kernel(
    _sc_bucket_body,
    out_type=(
        jax.ShapeDtypeStruct((NTILES * EC,), jnp.int32),
        jax.ShapeDtypeStruct((NTILES * EC,), jnp.int32),
        jax.ShapeDtypeStruct((NTILES * 16,), jnp.int32),
    ),
    mesh=plsc.VectorSubcoreMesh(core_axis_name="c", subcore_axis_name="s"),
    compiler_params=_PARAMS,
    scratch_types=[
        pltpu.VMEM((SCHUNK,), jnp.int32),
        pltpu.VMEM((SCHUNK,), jnp.int32),
        pltpu.VMEM((SCHUNK,), jnp.int32),
        pltpu.VMEM((SCHUNK,), jnp.int32),
        pltpu.VMEM((PENDC + 16,), jnp.int32),
        pltpu.VMEM((PENDC + 16,), jnp.int32),
        pltpu.VMEM((16,), jnp.int32),
        pltpu.SemaphoreType.DMA,
        pltpu.SemaphoreType.DMA,
    ],
)


WB = 21
WIN = WB * G


def _sc_agg_body(feat, bsrc, bdl, counts, agg_out,
                 srcw_v, dlw_v, cnt_v, rows0_v, rows1_v, rows2_v, acc_v,
                 deg_v, sem0, sem1, sem2):
    w = _wid()
    rowoff = w * EC

    zrow = jnp.full((16,), 0.0, jnp.float32)

    def zacc(r, c):
        for k in range(D // 16):
            acc_v[r, pl.ds(k * 16, 16)] = zrow
        return c

    lax.fori_loop(0, PT + 8, zacc, 0)
    for g in range((PT + 32) // 16):
        deg_v[pl.ds(g * 16, 16)] = zrow

    pltpu.sync_copy(counts.at[pl.ds(w * 16, 16)], cnt_v)
    cnt = cnt_v[pl.ds(0, 16)][0]
    nbatch = (cnt + G - 1) // G
    nwin = (nbatch + WB - 1) // WB
    onehot = jnp.where(lax.iota(jnp.int32, 16) == 0, 1.0, 0.0)
    iota16 = lax.iota(jnp.int32, 16)
    col_idx = [iota16 + 16 * k for k in range(D // 16)]

    def gdesc(ib, buf, sem):
        off = pl.multiple_of(ib * G, 8)
        return pltpu.make_async_copy(feat.at[srcw_v.at[pl.ds(off, G)]],
                                     buf, sem)

    def process(ib, buf):
        def edge(e, c):
            eidx = jnp.full((16,), 0, jnp.int32) + (ib * G + e)
            dlb = plsc.load_gather(dlw_v, [eidx])
            plsc.addupdate_scatter(deg_v, [dlb + iota16], onehot)
            vals = [buf[e, pl.ds(k * 16, 16)] for k in range(D // 16)]
            for k in range(D // 16):
                plsc.addupdate_scatter(acc_v, [dlb, col_idx[k]], vals[k])
            return c

        lax.fori_loop(0, G, edge, 0)

    def window(wi, carry):
        woff = pl.multiple_of(rowoff + wi * WIN, 8)
        pltpu.sync_copy(bsrc.at[pl.ds(woff, WIN)], srcw_v)
        pltpu.sync_copy(bdl.at[pl.ds(woff, WIN)], dlw_v.at[pl.ds(0, WIN)])
        nb = jnp.minimum(nbatch - wi * WB, WB)

        gdesc(0, rows0_v, sem0).start()

        @pl.when(nb >= 2)
        def _():
            gdesc(1, rows1_v, sem1).start()

        def triple(t, c):
            i = 3 * t
            gdesc(i, rows0_v, sem0).wait()

            @pl.when(i + 2 < nb)
            def _():
                gdesc(i + 2, rows2_v, sem2).start()

            process(i, rows0_v)
            gdesc(i + 1, rows1_v, sem1).wait()

            @pl.when(i + 3 < nb)
            def _():
                gdesc(i + 3, rows0_v, sem0).start()

            process(i + 1, rows1_v)
            gdesc(i + 2, rows2_v, sem2).wait()

            @pl.when(i + 4 < nb)
            def _():
                gdesc(i + 4, rows1_v, sem1).start()

            process(i + 2, rows2_v)
            return c

        ntrip = nb // 3
        lax.fori_loop(0, ntrip, triple, 0)
        rem = nb - 3 * ntrip

        @pl.when(rem >= 1)
        def _():
            gdesc(3 * ntrip, rows0_v, sem0).wait()
            process(3 * ntrip, rows0_v)

        @pl.when(rem == 2)
        def _():
            gdesc(3 * ntrip + 1, rows1_v, sem1).wait()
            process(3 * ntrip + 1, rows1_v)

        return carry

    lax.fori_loop(0, nwin, window, 0)

    def norm(r, c):
        recv = 1.0 / jnp.maximum(deg_v[pl.ds(r, 16)], 1.0)
        rec = recv[0]
        for k in range(D // 16):
            acc_v[r, pl.ds(k * 16, 16)] = acc_v[r, pl.ds(k * 16, 16)] * rec
        return c

    lax.fori_loop(0, PT, norm, 0)
    pltpu.sync_copy(acc_v.at[pl.ds(0, PT)], agg_out.at[pl.ds(w * PT, PT)])


_sc_agg = pl.kernel(
    _sc_agg_body,
    out_type=jax.ShapeDtypeStruct((NPAD, D), jnp.float32),
    mesh=plsc.VectorSubcoreMesh(core_axis_name="c", subcore_axis_name="s"),
    compiler_params=_PARAMS,
    scratch_types=[
        pltpu.VMEM((WIN,), jnp.int32),
        pltpu.VMEM((WIN + 16,), jnp.int32),
        pltpu.VMEM((16,), jnp.int32),
        pltpu.VMEM((G, D), jnp.float32),
        pltpu.VMEM((G, D), jnp.float32),
        pltpu.VMEM((G, D), jnp.float32),
        pltpu.VMEM((PT + 8, D), jnp.float32),
        pltpu.VMEM((PT + 32,), jnp.float32),
        pltpu.SemaphoreType.DMA,
        pltpu.SemaphoreType.DMA,
        pltpu.SemaphoreType.DMA,
    ],
)


def _dense_body(agg_ref, x_ref, wl_ref, wr_ref, b_ref, o_ref, *, relu):
    acc = lax.dot_general(agg_ref[...], wl_ref[...], (((1,), (1,)), ((), ())),
                          preferred_element_type=jnp.float32)
    acc = acc + lax.dot_general(x_ref[...], wr_ref[...], (((1,), (1,)), ((), ())),
                                preferred_element_type=jnp.float32)
    acc = acc + b_ref[...]
    if relu:
        acc = jnp.maximum(acc, 0.0)
    o_ref[...] = acc


def _dense(aggn, x, W_l, W_r, b, relu):
    BR = 1024
    return pl.pallas_call(
        functools.partial(_dense_body, relu=relu),
        grid=(NPAD // BR,),
        in_specs=[
            pl.BlockSpec((BR, D), lambda i: (i, 0)),
            pl.BlockSpec((BR, D), lambda i: (i, 0)),
            pl.BlockSpec((D, D), lambda i: (0, 0)),
            pl.BlockSpec((D, D), lambda i: (0, 0)),
            pl.BlockSpec((1, D), lambda i: (0, 0)),
        ],
        out_specs=pl.BlockSpec((BR, D), lambda i: (i, 0)),
        out_shape=jax.ShapeDtypeStruct((NPAD, D), jnp.float32),
    )(aggn, x, W_l, W_r, b)


def kernel(x, edge_index, W1_l, b1_l, W1_r, W2_l, b2_l, W2_r):
    x = x.astype(jnp.float32)
    ei = edge_index.astype(jnp.int32)
    src = ei[0]
    dst = ei[1]
    xpad = jnp.zeros((NPAD, D), jnp.float32).at[:N].set(x)
    b1 = b1_l.reshape(1, D)
    b2 = b2_l.reshape(1, D)

    bsrc, bdl, counts = _sc_bucket(src, dst)
    agg1 = _sc_agg(xpad, bsrc, bdl, counts)
    h = _dense(agg1, xpad, W1_l, W1_r, b1, relu=True)
    agg2 = _sc_agg(h, bsrc, bdl, counts)
    out = _dense(agg2, h, W2_l, W2_r, b2, relu=False)
    return out[:N]
